# baseline (speedup 1.0000x reference)
import functools

import jax
import jax.numpy as jnp
from jax import lax
from jax.experimental import pallas as pl
from jax.experimental.pallas import tpu as pltpu
from jax.experimental.pallas import tpu_sc as plsc

B = 4096
L = 200
D = 64
DP = 128
SRC_MAX_LEN = 200
NW = 32
B_PER_W = B // NW
IDXG = 40
NIDXG = L // IDXG
SCALE = float(D) ** 0.5

_mesh = plsc.VectorSubcoreMesh(core_axis_name="c", subcore_axis_name="s")


@functools.partial(
    pl.kernel,
    mesh=_mesh,
    out_type=jax.ShapeDtypeStruct((B, L, D), jnp.float32),
    scratch_types=[
        pltpu.VMEM((L, D), jnp.float32),
        pltpu.VMEM((L, DP), jnp.float32),
        pltpu.VMEM((L,), jnp.int32),
        pltpu.SemaphoreType.DMA,
    ],
)
def _sc_kernel(src_hbm, pos_hbm, emb_hbm, out_hbm, a_v, g_v, idx_v, sem):
    wid = lax.axis_index("s") * 2 + lax.axis_index("c")
    base = wid * B_PER_W

    def chunk_body(ci, carry):
        b0 = base + ci
        pltpu.sync_copy(src_hbm.at[b0], a_v)
        pltpu.sync_copy(pos_hbm.at[pl.ds(b0 * L, L)], idx_v)
        cps = [
            pltpu.async_copy(
                emb_hbm.at[idx_v.at[pl.ds(j * IDXG, IDXG)]],
                g_v.at[pl.ds(j * IDXG, IDXG)],
                sem,
            )
            for j in range(NIDXG)
        ]
        for cp in cps:
            cp.wait()

        def row_body(r, c2):
            for j in range(D // 16):
                s = pl.ds(j * 16, 16)
                a_v[r, s] = a_v[r, s] * SCALE + g_v[r, s]
            return c2

        lax.fori_loop(0, L, row_body, 0, unroll=2)
        pltpu.sync_copy(a_v, out_hbm.at[b0])
        return carry

    lax.fori_loop(0, B_PER_W, chunk_body, 0)


def kernel(src_word, src_pos, emb):
    pos = src_pos.reshape(B * L).astype(jnp.int32)
    emb_p = jnp.pad(emb.astype(jnp.float32), ((0, 0), (0, DP - D)))
    out = _sc_kernel(src_word.astype(jnp.float32), pos, emb_p)
    return out


# speedup vs baseline: 1.1307x; 1.1307x over previous
import functools

import jax
import jax.numpy as jnp
from jax import lax
from jax.experimental import pallas as pl
from jax.experimental.pallas import tpu as pltpu
from jax.experimental.pallas import tpu_sc as plsc

B = 4096
L = 200
D = 64
DP = 128
NW = 32
B_PER_W = B // NW
IDXG = 128
SCALE = float(D) ** 0.5

_mesh = plsc.VectorSubcoreMesh(core_axis_name="c", subcore_axis_name="s")


@functools.partial(
    pl.kernel,
    mesh=_mesh,
    out_type=jax.ShapeDtypeStruct((B, L, D), jnp.float32),
    scratch_types=[
        [pltpu.VMEM((L, D), jnp.float32)] * 2,
        [pltpu.VMEM((L, DP), jnp.float32)] * 2,
        [pltpu.VMEM((L,), jnp.int32)] * 2,
        [pltpu.SemaphoreType.DMA] * 2,
        [pltpu.SemaphoreType.DMA] * 2,
        [pltpu.SemaphoreType.DMA] * 2,
        [pltpu.SemaphoreType.DMA] * 2,
    ],
)
def _sc_kernel(src_hbm, pos_hbm, emb_hbm, out_hbm, a_v, g_v, idx_v,
               s_src, s_idx, s_g, s_out):
    wid = lax.axis_index("s") * 2 + lax.axis_index("c")
    base = wid * B_PER_W

    def src_cp(ci, k):
        return pltpu.make_async_copy(src_hbm.at[base + ci], a_v[k],
                                     s_src[k])

    def idx_cp(ci, k):
        return pltpu.make_async_copy(pos_hbm.at[pl.ds((base + ci) * L, L)],
                                     idx_v[k], s_idx[k])

    def gather_cps(k):
        return [
            pltpu.make_async_copy(
                emb_hbm.at[idx_v[k].at[pl.ds(0, IDXG)]],
                g_v[k].at[pl.ds(0, IDXG)], s_g[k]),
            pltpu.make_async_copy(
                emb_hbm.at[idx_v[k].at[pl.ds(IDXG, L - IDXG)]],
                g_v[k].at[pl.ds(IDXG, L - IDXG)], s_g[k]),
        ]

    def out_cp(ci, k):
        return pltpu.make_async_copy(a_v[k], out_hbm.at[base + ci],
                                     s_out[k])

    idx_cp(0, 0).start()
    idx_cp(1, 1).start()
    src_cp(0, 0).start()
    src_cp(1, 1).start()
    idx_cp(0, 0).wait()
    for cp in gather_cps(0):
        cp.start()

    def body(si, carry):
        for cur in range(2):
            ci = si * 2 + cur
            nxt = 1 - cur

            @pl.when(ci + 1 < B_PER_W)
            def _(nxt=nxt, ci=ci):
                idx_cp(ci + 1, nxt).wait()
                for cp in gather_cps(nxt):
                    cp.start()

            src_cp(ci, cur).wait()
            gather_cps(cur)[0].wait()
            gather_cps(cur)[1].wait()

            def row_body(r, c2, cur=cur):
                av, gv = a_v[cur], g_v[cur]
                for j in range(D // 16):
                    s = pl.ds(j * 16, 16)
                    av[r, s] = av[r, s] * SCALE + gv[r, s]
                return c2

            lax.fori_loop(0, L, row_body, 0, unroll=2)
            out_cp(ci, cur).start()

            @pl.when(ci + 2 < B_PER_W)
            def _(cur=cur, ci=ci):
                idx_cp(ci + 2, cur).start()
                out_cp(ci, cur).wait()
                src_cp(ci + 2, cur).start()

            @pl.when(ci + 2 >= B_PER_W)
            def _(cur=cur, ci=ci):
                out_cp(ci, cur).wait()

        return carry

    lax.fori_loop(0, B_PER_W // 2, body, 0)


def kernel(src_word, src_pos, emb):
    pos = src_pos.reshape(B * L).astype(jnp.int32)
    emb_p = jnp.pad(emb.astype(jnp.float32), ((0, 0), (0, DP - D)))
    out = _sc_kernel(src_word.astype(jnp.float32), pos, emb_p)
    return out


# speedup vs baseline: 1.1391x; 1.0074x over previous
import functools

import jax
import jax.numpy as jnp
from jax import lax
from jax.experimental import pallas as pl
from jax.experimental.pallas import tpu as pltpu
from jax.experimental.pallas import tpu_sc as plsc

B = 4096
L = 200
D = 64
DP = 128
NW = 32
B_PER_W = B // NW
IDXG = 128
SCALE = float(D) ** 0.5

_mesh = plsc.VectorSubcoreMesh(core_axis_name="c", subcore_axis_name="s")


@functools.partial(
    pl.kernel,
    mesh=_mesh,
    out_type=jax.ShapeDtypeStruct((B, L, D), jnp.float32),
    scratch_types=[
        [pltpu.VMEM((L, D), jnp.float32)] * 2,
        [pltpu.VMEM((L, DP), jnp.float32)] * 2,
        [pltpu.VMEM((L,), jnp.int32)] * 2,
        [pltpu.SemaphoreType.DMA] * 2,
        [pltpu.SemaphoreType.DMA] * 2,
        [pltpu.SemaphoreType.DMA] * 2,
        [pltpu.SemaphoreType.DMA] * 2,
    ],
)
def _sc_kernel(src_hbm, pos_hbm, emb_hbm, out_hbm, a_v, g_v, idx_v,
               s_src, s_idx, s_g, s_out):
    wid = lax.axis_index("s") * 2 + lax.axis_index("c")
    base = wid * B_PER_W

    def src_cp(ci, k):
        return pltpu.make_async_copy(src_hbm.at[base + ci], a_v[k],
                                     s_src[k])

    def idx_cp(ci, k):
        return pltpu.make_async_copy(pos_hbm.at[pl.ds((base + ci) * L, L)],
                                     idx_v[k], s_idx[k])

    def gather_cps(k):
        return [
            pltpu.make_async_copy(
                emb_hbm.at[idx_v[k].at[pl.ds(0, IDXG)]],
                g_v[k].at[pl.ds(0, IDXG)], s_g[k]),
            pltpu.make_async_copy(
                emb_hbm.at[idx_v[k].at[pl.ds(IDXG, L - IDXG)]],
                g_v[k].at[pl.ds(IDXG, L - IDXG)], s_g[k]),
        ]

    def out_cp(ci, k):
        return pltpu.make_async_copy(a_v[k], out_hbm.at[base + ci],
                                     s_out[k])

    idx_cp(0, 0).start()
    idx_cp(1, 1).start()
    src_cp(0, 0).start()
    src_cp(1, 1).start()
    idx_cp(0, 0).wait()
    for cp in gather_cps(0):
        cp.start()

    def body(si, carry):
        for cur in range(2):
            ci = si * 2 + cur
            nxt = 1 - cur

            @pl.when(ci + 1 < B_PER_W)
            def _(nxt=nxt, ci=ci):
                idx_cp(ci + 1, nxt).wait()
                for cp in gather_cps(nxt):
                    cp.start()

            src_cp(ci, cur).wait()
            gather_cps(cur)[0].wait()
            gather_cps(cur)[1].wait()

            av, gv = a_v[cur], g_v[cur]

            @plsc.parallel_loop(0, L, unroll=4)
            def _row(r):
                for j in range(D // 16):
                    s = pl.ds(j * 16, 16)
                    av[r, s] = av[r, s] * SCALE + gv[r, s]
            out_cp(ci, cur).start()

            @pl.when(ci + 2 < B_PER_W)
            def _(cur=cur, ci=ci):
                idx_cp(ci + 2, cur).start()
                out_cp(ci, cur).wait()
                src_cp(ci + 2, cur).start()

            @pl.when(ci + 2 >= B_PER_W)
            def _(cur=cur, ci=ci):
                out_cp(ci, cur).wait()

        return carry

    lax.fori_loop(0, B_PER_W // 2, body, 0)


def kernel(src_word, src_pos, emb):
    pos = src_pos.reshape(B * L).astype(jnp.int32)
    emb_p = jnp.pad(emb.astype(jnp.float32), ((0, 0), (0, DP - D)))
    out = _sc_kernel(src_word.astype(jnp.float32), pos, emb_p)
    return out


# speedup vs baseline: 1.1394x; 1.0003x over previous
import functools

import jax
import jax.numpy as jnp
from jax import lax
from jax.experimental import pallas as pl
from jax.experimental.pallas import tpu as pltpu
from jax.experimental.pallas import tpu_sc as plsc

B = 4096
L = 200
D = 64
DP = 128
NW = 32
B_PER_W = B // NW
IDXG = 128
SCALE = float(D) ** 0.5

_mesh = plsc.VectorSubcoreMesh(core_axis_name="c", subcore_axis_name="s")


@functools.partial(
    pl.kernel,
    mesh=_mesh,
    out_type=jax.ShapeDtypeStruct((B, L, D), jnp.float32),
    scratch_types=[
        [pltpu.VMEM((L + 8, D), jnp.float32)] * 2,
        [pltpu.VMEM((L + 8, DP), jnp.float32)] * 2,
        [pltpu.VMEM((L,), jnp.int32)] * 2,
        [pltpu.SemaphoreType.DMA] * 2,
        [pltpu.SemaphoreType.DMA] * 2,
        [pltpu.SemaphoreType.DMA] * 2,
        [pltpu.SemaphoreType.DMA] * 2,
    ],
)
def _sc_kernel(src_hbm, pos_hbm, emb_hbm, out_hbm, a_v, g_v, idx_v,
               s_src, s_idx, s_g, s_out):
    wid = lax.axis_index("s") * 2 + lax.axis_index("c")
    base = wid * B_PER_W

    def src_cp(ci, k):
        return pltpu.make_async_copy(src_hbm.at[base + ci],
                                     a_v[k].at[pl.ds(0, L)], s_src[k])

    def idx_cp(ci, k):
        return pltpu.make_async_copy(pos_hbm.at[pl.ds((base + ci) * L, L)],
                                     idx_v[k], s_idx[k])

    def gather_cps(k):
        return [
            pltpu.make_async_copy(
                emb_hbm.at[idx_v[k].at[pl.ds(0, IDXG)]],
                g_v[k].at[pl.ds(0, IDXG)], s_g[k]),
            pltpu.make_async_copy(
                emb_hbm.at[idx_v[k].at[pl.ds(IDXG, L - IDXG)]],
                g_v[k].at[pl.ds(IDXG, L - IDXG)], s_g[k]),
        ]

    def out_cp(ci, k):
        return pltpu.make_async_copy(a_v[k].at[pl.ds(0, L)],
                                     out_hbm.at[base + ci], s_out[k])

    idx_cp(0, 0).start()
    idx_cp(1, 1).start()
    src_cp(0, 0).start()
    src_cp(1, 1).start()
    idx_cp(0, 0).wait()
    for cp in gather_cps(0):
        cp.start()

    def body(si, carry):
        for cur in range(2):
            ci = si * 2 + cur
            nxt = 1 - cur

            @pl.when(ci + 1 < B_PER_W)
            def _(nxt=nxt, ci=ci):
                idx_cp(ci + 1, nxt).wait()
                for cp in gather_cps(nxt):
                    cp.start()

            src_cp(ci, cur).wait()
            gather_cps(cur)[0].wait()
            gather_cps(cur)[1].wait()

            av, gv = a_v[cur], g_v[cur]

            @plsc.parallel_loop(0, L, unroll=4)
            def _row(r):
                for j in range(D // 16):
                    s = pl.ds(j * 16, 16)
                    av[r, s] = av[r, s] * SCALE + gv[r, s]
            out_cp(ci, cur).start()

            @pl.when(ci + 2 < B_PER_W)
            def _(cur=cur, ci=ci):
                idx_cp(ci + 2, cur).start()
                out_cp(ci, cur).wait()
                src_cp(ci + 2, cur).start()

            @pl.when(ci + 2 >= B_PER_W)
            def _(cur=cur, ci=ci):
                out_cp(ci, cur).wait()

        return carry

    lax.fori_loop(0, B_PER_W // 2, body, 0)


def kernel(src_word, src_pos, emb):
    pos = src_pos.reshape(B * L).astype(jnp.int32)
    emb_p = jnp.pad(emb.astype(jnp.float32), ((0, 0), (0, DP - D)))
    out = _sc_kernel(src_word.astype(jnp.float32), pos, emb_p)
    return out


# speedup vs baseline: 1.3908x; 1.2207x over previous
import functools

import jax
import jax.numpy as jnp
from jax import lax
from jax.experimental import pallas as pl
from jax.experimental.pallas import tpu as pltpu
from jax.experimental.pallas import tpu_sc as plsc

D = 64
SRC_MAX_LEN = 200
R = 4096 * 200
NW = 32
ROWS_PER_W = R // NW
CHUNK = 512
NCHUNK = ROWS_PER_W // CHUNK
IDXG = 128
NIDXG = CHUNK // IDXG
SCALE = float(D) ** 0.5

_mesh = plsc.VectorSubcoreMesh(core_axis_name="c", subcore_axis_name="s")


@functools.partial(
    pl.kernel,
    mesh=_mesh,
    out_type=jax.ShapeDtypeStruct((R, D), jnp.float32),
    compiler_params=pltpu.CompilerParams(use_tc_tiling_on_sc=False),
    scratch_types=[
        pltpu.VMEM((CHUNK + 8, D), jnp.float32),
        pltpu.VMEM((SRC_MAX_LEN, D), jnp.float32),
        pltpu.VMEM_SHARED((SRC_MAX_LEN, D), jnp.float32),
        pltpu.VMEM((CHUNK,), jnp.int32),
        pltpu.SemaphoreType.DMA,
    ],
)
def _sc_kernel(src_hbm, pos_hbm, emb_hbm, out_hbm, a_v, stage_v, table_s,
               idx_v, sem):
    wid = lax.axis_index("s") * 2 + lax.axis_index("c")
    base = wid * ROWS_PER_W

    pltpu.sync_copy(emb_hbm, stage_v)

    @plsc.parallel_loop(0, SRC_MAX_LEN)
    def _scale(r):
        for j in range(D // 16):
            s = pl.ds(j * 16, 16)
            stage_v[r, s] = stage_v[r, s] * (1.0 / SCALE)

    pltpu.sync_copy(stage_v, table_s)

    def chunk_body(ci, carry):
        row0 = base + ci * CHUNK
        pltpu.sync_copy(src_hbm.at[pl.ds(row0, CHUNK)],
                        a_v.at[pl.ds(0, CHUNK)])
        pltpu.sync_copy(pos_hbm.at[pl.ds(row0, CHUNK)], idx_v)
        cps = [
            pltpu.async_copy(
                table_s.at[idx_v.at[pl.ds(j * IDXG, IDXG)]],
                a_v.at[pl.ds(j * IDXG, IDXG)],
                sem,
                add=True,
            )
            for j in range(NIDXG)
        ]
        for cp in cps:
            cp.wait()

        @plsc.parallel_loop(0, CHUNK, unroll=4)
        def _row(r):
            for j in range(D // 16):
                s = pl.ds(j * 16, 16)
                a_v[r, s] = a_v[r, s] * SCALE

        pltpu.sync_copy(a_v.at[pl.ds(0, CHUNK)],
                        out_hbm.at[pl.ds(row0, CHUNK)])
        return carry

    lax.fori_loop(0, NCHUNK, chunk_body, 0)


def kernel(src_word, src_pos, emb):
    src = src_word.reshape(R, D).astype(jnp.float32)
    pos = src_pos.reshape(R).astype(jnp.int32)
    out = _sc_kernel(src, pos, emb.astype(jnp.float32))
    return out.reshape(src_word.shape)
